# baseline (speedup 1.0000x reference)
import functools

import numpy as np

import jax
import jax.numpy as jnp
from jax import lax
from jax.experimental import pallas as pl
from jax.experimental.pallas import tpu as pltpu
from jax.experimental.pallas import tpu_sc as plsc

_NC = 2
_NS = 16
_L = 16
_NW = _NC * _NS
_C = 64
_B = 32
_NBUF = 3


def _cc1(x):
    return ((-0.75 + 2.0) * x - (-0.75 + 3.0)) * x * x + 1.0


def _cc2(x):
    return ((-0.75 * x - 5.0 * -0.75) * x + 8.0 * -0.75) * x - 4.0 * -0.75


def _make_sc_call(P, grids):
    pts_w = P // _NW
    nchunk = pts_w // _B
    mesh = plsc.VectorSubcoreMesh(
        core_axis_name="c", subcore_axis_name="s",
        num_cores=_NC, num_subcores=_NS)

    def body(ct, t1x, t1y, t1z, t2x, t2y, t2z, t3x, t3y, t3z,
             o1, o2, o3, cb, ti, tw, idx, wc, rows, acc,
             sem0, sem1, sem2):
        wid = lax.axis_index("s") * _NC + lax.axis_index("c")
        start = wid * pts_w
        sems = (sem0, sem1, sem2)
        scales = ((grids[0], (t1x, t1y, t1z), o1),
                  (grids[1], (t2x, t2y, t2z), o2),
                  (grids[2], (t3x, t3y, t3z), o3))
        axes = ((0, 1), (1, 2), (0, 2))

        def load_coords(ci):
            base = start + ci * _B
            for k in range(3):
                pltpu.sync_copy(ct.at[k, pl.ds(base, _B)], cb.at[k])

        def comp_taps(W):
            for k in range(3):
                for v in range(_B // _L):
                    sl = pl.ds(v * _L, _L)
                    c = (cb[k, sl] + 1.0) * 0.5
                    ix = (c + 1.0) * 0.5 * (W - 1)
                    f = ix.astype(jnp.int32)
                    t = ix - f.astype(jnp.float32)
                    ws = (_cc2(t + 1.0), _cc1(t), _cc1(1.0 - t),
                          _cc2(2.0 - t))
                    for j in range(4):
                        ti[k, j, sl] = jnp.clip(f - 1 + j, 0, W - 1)
                        tw[k, j, sl] = ws[j]

        def build_fire(par, step):
            s, p = step // 3, step % 3
            W, tbls, _ = scales[s]
            a, b = axes[p]
            tbl = tbls[p]
            for j in range(4):
                for i in range(4):
                    t = j * 4 + i
                    for v in range(_B // _L):
                        sl = pl.ds(v * _L, _L)
                        idx[par, t, sl] = ti[b, j, sl] * W + ti[a, i, sl]
            for j in range(4):
                for v in range(_B // _L):
                    sl = pl.ds(v * _L, _L)
                    wc[par, j, sl] = tw[b, j, sl]
                    wc[par, 4 + j, sl] = tw[a, j, sl]
            for t in range(16):
                pltpu.async_copy(tbl.at[idx.at[par, t]], rows.at[par, t],
                                 sems[par])

        def wait_step(par, step):
            tbl = scales[step // 3][1][step % 3]
            for t in range(16):
                pltpu.make_async_copy(tbl.at[idx.at[par, t]],
                                      rows.at[par, t], sems[par]).wait()

        def accumulate(par, first):
            def point(p, c2):
                wy = [wc[par, j, pl.ds(p, _L)][0] for j in range(4)]
                wx = [wc[par, 4 + i, pl.ds(p, _L)][0] for i in range(4)]
                for g in range(_C // (2 * _L)):
                    ga = pl.ds(g * 2 * _L, _L)
                    gb = pl.ds(g * 2 * _L + _L, _L)
                    ava = None if first else acc[p, ga]
                    avb = None if first else acc[p, gb]
                    for j in range(4):
                        ba = bb = None
                        for i in range(4):
                            x = rows[par, j * 4 + i, p,
                                     pl.ds(g * 2 * _L, 2 * _L)]
                            a, b = plsc.unpack(
                                x, format=plsc.PackFormat.INTERLEAVED)
                            wa, wb2 = wx[i] * a, wx[i] * b
                            ba = wa if ba is None else ba + wa
                            bb = wb2 if bb is None else bb + wb2
                        pa, pb = wy[j] * ba, wy[j] * bb
                        ava = pa if ava is None else ava + pa
                        avb = pb if avb is None else avb + pb
                    acc[p, ga] = ava
                    acc[p, gb] = avb
                return c2
            lax.fori_loop(0, _B, point, 0)

        load_coords(0)
        comp_taps(scales[0][0])
        build_fire(0, 0)

        def chunk(ci, carry):
            base = start + ci * _B
            for k in range(9):
                par = k % _NBUF
                nxt = (k + 1) % _NBUF
                if k < 8:
                    if (k + 1) % 3 == 0:
                        comp_taps(scales[(k + 1) // 3][0])
                    build_fire(nxt, k + 1)
                else:
                    @pl.when(ci < nchunk - 1)
                    def _():
                        load_coords(ci + 1)
                        comp_taps(scales[0][0])
                        build_fire(nxt, 0)
                wait_step(par, k)
                accumulate(par, first=(k % 3 == 0))
                if k % 3 == 2:
                    out = scales[k // 3][2]
                    pltpu.sync_copy(acc, out.at[pl.ds(base, _B)])
            return carry
        lax.fori_loop(0, nchunk, chunk, 0)

    out_type = [jax.ShapeDtypeStruct((P, _C), jnp.float32)
                for _ in range(3)]
    scratch = [
        pltpu.VMEM((3, _B), jnp.float32),
        pltpu.VMEM((3, 4, _B), jnp.int32),
        pltpu.VMEM((3, 4, _B), jnp.float32),
        pltpu.VMEM((_NBUF, 16, _B), jnp.int32),
        pltpu.VMEM((_NBUF, 8, _B + _L), jnp.float32),
        pltpu.VMEM((_NBUF, 16, _B, _C), jnp.bfloat16),
        pltpu.VMEM((_B, _C), jnp.float32),
        pltpu.SemaphoreType.DMA,
        pltpu.SemaphoreType.DMA,
        pltpu.SemaphoreType.DMA,
    ]
    return pl.kernel(body, out_type=out_type, mesh=mesh,
                     scratch_types=scratch,
                     compiler_params=pltpu.CompilerParams(
                         use_tc_tiling_on_sc=False,
                         needs_layout_passes=False))


def kernel(coordinates, px1, py1, pz1, px2, py2, pz2, px3, py3, pz3):
    P = coordinates.shape[0]
    planes = (px1, py1, pz1, px2, py2, pz2, px3, py3, pz3)
    grids = (px1.shape[1], px2.shape[1], px3.shape[1])
    ct = coordinates.T
    perm = np.stack([np.arange(16), np.arange(16) + 16], 1).reshape(-1)
    perm = np.concatenate([perm, perm + 32])
    tables = [
        jnp.transpose(p, (1, 2, 0)).reshape(-1, _C)
        .astype(jnp.bfloat16)[:, perm]
        for p in planes]
    call = _make_sc_call(P, grids)
    o1, o2, o3 = call(ct, *tables)
    return jnp.concatenate([o1, o2, o3], axis=-1)


# speedup vs baseline: 1.1239x; 1.1239x over previous
import functools

import numpy as np

import jax
import jax.numpy as jnp
from jax import lax
from jax.experimental import pallas as pl
from jax.experimental.pallas import tpu as pltpu
from jax.experimental.pallas import tpu_sc as plsc

_NC = 2
_NS = 16
_L = 16
_NW = _NC * _NS
_C = 64
_B = 32
_NBUF = 3


def _cc1(x):
    return ((-0.75 + 2.0) * x - (-0.75 + 3.0)) * x * x + 1.0


def _cc2(x):
    return ((-0.75 * x - 5.0 * -0.75) * x + 8.0 * -0.75) * x - 4.0 * -0.75


def _make_sc_call(P, grids):
    pts_w = P // _NW
    nchunk = pts_w // _B
    mesh = plsc.VectorSubcoreMesh(
        core_axis_name="c", subcore_axis_name="s",
        num_cores=_NC, num_subcores=_NS)

    def body(ct, t1x, t1y, t1z, t2x, t2y, t2z, t3x, t3y, t3z,
             o1, o2, o3, cb, ti, tw, idx, wc, rows, acc,
             sem0, sem1, sem2):
        wid = lax.axis_index("s") * _NC + lax.axis_index("c")
        start = wid * pts_w
        sems = (sem0, sem1, sem2)
        scales = ((grids[0], (t1x, t1y, t1z), o1),
                  (grids[1], (t2x, t2y, t2z), o2),
                  (grids[2], (t3x, t3y, t3z), o3))
        axes = ((0, 1), (1, 2), (0, 2))

        def load_coords(ci):
            base = start + ci * _B
            for k in range(3):
                pltpu.sync_copy(ct.at[k, pl.ds(base, _B)], cb.at[k])

        def comp_taps(W):
            for k in range(3):
                for v in range(_B // _L):
                    sl = pl.ds(v * _L, _L)
                    c = (cb[k, sl] + 1.0) * 0.5
                    ix = (c + 1.0) * 0.5 * (W - 1)
                    f = ix.astype(jnp.int32)
                    t = ix - f.astype(jnp.float32)
                    ws = (_cc2(t + 1.0), _cc1(t), _cc1(1.0 - t),
                          _cc2(2.0 - t))
                    for j in range(4):
                        ti[k, j, sl] = jnp.clip(f - 1 + j, 0, W - 1)
                        tw[k, j, sl] = ws[j]

        def build_fire(par, step):
            s, p = step // 3, step % 3
            W, tbls, _ = scales[s]
            a, b = axes[p]
            tbl = tbls[p]
            for j in range(4):
                for i in range(4):
                    t = j * 4 + i
                    for v in range(_B // _L):
                        sl = pl.ds(v * _L, _L)
                        idx[par, t, sl] = ti[b, j, sl] * W + ti[a, i, sl]
            for j in range(4):
                for v in range(_B // _L):
                    sl = pl.ds(v * _L, _L)
                    wc[par, j, sl] = tw[b, j, sl]
                    wc[par, 4 + j, sl] = tw[a, j, sl]
            for t in range(16):
                pltpu.async_copy(tbl.at[idx.at[par, t]], rows.at[par, t],
                                 sems[par])

        def wait_step(par, step):
            tbl = scales[step // 3][1][step % 3]
            for t in range(16):
                pltpu.make_async_copy(tbl.at[idx.at[par, t]],
                                      rows.at[par, t], sems[par]).wait()

        def accumulate(par, first):
            ones = jnp.ones((_L,), jnp.float32)

            def point(p, c2):
                wy = [wc[par, j, pl.ds(p, _L)][0] for j in range(4)]
                wx = [wc[par, 4 + i, pl.ds(p, _L)][0] for i in range(4)]
                wxp = [plsc.pack(w * ones, w * ones,
                                 format=plsc.PackFormat.INTERLEAVED)
                       for w in wx]
                for g in range(_C // (2 * _L)):
                    ga = pl.ds(g * 2 * _L, _L)
                    gb = pl.ds(g * 2 * _L + _L, _L)
                    ava = None if first else acc[p, ga]
                    avb = None if first else acc[p, gb]
                    for j in range(4):
                        bv = None
                        for i in range(4):
                            x = rows[par, j * 4 + i, p,
                                     pl.ds(g * 2 * _L, 2 * _L)]
                            t = wxp[i] * x
                            bv = t if bv is None else bv + t
                        a, b = plsc.unpack(
                            bv, format=plsc.PackFormat.INTERLEAVED)
                        pa, pb = wy[j] * a, wy[j] * b
                        ava = pa if ava is None else ava + pa
                        avb = pb if avb is None else avb + pb
                    acc[p, ga] = ava
                    acc[p, gb] = avb
                return c2
            lax.fori_loop(0, _B, point, 0)

        load_coords(0)
        comp_taps(scales[0][0])
        build_fire(0, 0)

        def chunk(ci, carry):
            base = start + ci * _B
            for k in range(9):
                par = k % _NBUF
                nxt = (k + 1) % _NBUF
                if k < 8:
                    if (k + 1) % 3 == 0:
                        comp_taps(scales[(k + 1) // 3][0])
                    build_fire(nxt, k + 1)
                else:
                    @pl.when(ci < nchunk - 1)
                    def _():
                        load_coords(ci + 1)
                        comp_taps(scales[0][0])
                        build_fire(nxt, 0)
                wait_step(par, k)
                accumulate(par, first=(k % 3 == 0))
                if k % 3 == 2:
                    out = scales[k // 3][2]
                    pltpu.sync_copy(acc, out.at[pl.ds(base, _B)])
            return carry
        lax.fori_loop(0, nchunk, chunk, 0)

    out_type = [jax.ShapeDtypeStruct((P, _C), jnp.float32)
                for _ in range(3)]
    scratch = [
        pltpu.VMEM((3, _B), jnp.float32),
        pltpu.VMEM((3, 4, _B), jnp.int32),
        pltpu.VMEM((3, 4, _B), jnp.float32),
        pltpu.VMEM((_NBUF, 16, _B), jnp.int32),
        pltpu.VMEM((_NBUF, 8, _B + _L), jnp.float32),
        pltpu.VMEM((_NBUF, 16, _B, _C), jnp.bfloat16),
        pltpu.VMEM((_B, _C), jnp.float32),
        pltpu.SemaphoreType.DMA,
        pltpu.SemaphoreType.DMA,
        pltpu.SemaphoreType.DMA,
    ]
    return pl.kernel(body, out_type=out_type, mesh=mesh,
                     scratch_types=scratch,
                     compiler_params=pltpu.CompilerParams(
                         use_tc_tiling_on_sc=False,
                         needs_layout_passes=False))


def kernel(coordinates, px1, py1, pz1, px2, py2, pz2, px3, py3, pz3):
    P = coordinates.shape[0]
    planes = (px1, py1, pz1, px2, py2, pz2, px3, py3, pz3)
    grids = (px1.shape[1], px2.shape[1], px3.shape[1])
    ct = coordinates.T
    perm = np.stack([np.arange(16), np.arange(16) + 16], 1).reshape(-1)
    perm = np.concatenate([perm, perm + 32])
    tables = [
        jnp.transpose(p, (1, 2, 0)).reshape(-1, _C)
        .astype(jnp.bfloat16)[:, perm]
        for p in planes]
    call = _make_sc_call(P, grids)
    o1, o2, o3 = call(ct, *tables)
    return jnp.concatenate([o1, o2, o3], axis=-1)


# speedup vs baseline: 1.1900x; 1.0588x over previous
import functools

import numpy as np

import jax
import jax.numpy as jnp
from jax import lax
from jax.experimental import pallas as pl
from jax.experimental.pallas import tpu as pltpu
from jax.experimental.pallas import tpu_sc as plsc

_NC = 2
_NS = 16
_L = 16
_NW = _NC * _NS
_C = 64
_B = 32
_NBUF = 3


def _cc1(x):
    return ((-0.75 + 2.0) * x - (-0.75 + 3.0)) * x * x + 1.0


def _cc2(x):
    return ((-0.75 * x - 5.0 * -0.75) * x + 8.0 * -0.75) * x - 4.0 * -0.75


def _make_sc_call(P, grids):
    pts_w = P // _NW
    nchunk = pts_w // _B
    mesh = plsc.VectorSubcoreMesh(
        core_axis_name="c", subcore_axis_name="s",
        num_cores=_NC, num_subcores=_NS)

    def body(ct, t1x, t1y, t1z, t2x, t2y, t2z, t3x, t3y, t3z,
             out, cb, ti, tw, idx, wc, rows, acc,
             sem0, sem1, sem2):
        wid = lax.axis_index("s") * _NC + lax.axis_index("c")
        start = wid * pts_w
        sems = (sem0, sem1, sem2)
        scales = ((grids[0], (t1x, t1y, t1z), 0),
                  (grids[1], (t2x, t2y, t2z), _C),
                  (grids[2], (t3x, t3y, t3z), 2 * _C))
        axes = ((0, 1), (1, 2), (0, 2))

        def load_coords(ci):
            base = start + ci * _B
            for k in range(3):
                pltpu.sync_copy(ct.at[k, pl.ds(base, _B)], cb.at[k])

        def comp_taps(W):
            for k in range(3):
                for v in range(_B // _L):
                    sl = pl.ds(v * _L, _L)
                    c = (cb[k, sl] + 1.0) * 0.5
                    ix = (c + 1.0) * 0.5 * (W - 1)
                    f = ix.astype(jnp.int32)
                    t = ix - f.astype(jnp.float32)
                    ws = (_cc2(t + 1.0), _cc1(t), _cc1(1.0 - t),
                          _cc2(2.0 - t))
                    for j in range(4):
                        ti[k, j, sl] = jnp.clip(f - 1 + j, 0, W - 1)
                        tw[k, j, sl] = ws[j]

        def build_fire(par, step):
            s, p = step // 3, step % 3
            W, tbls, _ = scales[s]
            a, b = axes[p]
            tbl = tbls[p]
            for j in range(4):
                for i in range(4):
                    t = j * 4 + i
                    for v in range(_B // _L):
                        sl = pl.ds(v * _L, _L)
                        idx[par, t, sl] = ti[b, j, sl] * W + ti[a, i, sl]
            for j in range(4):
                for v in range(_B // _L):
                    sl = pl.ds(v * _L, _L)
                    wc[par, j, sl] = tw[b, j, sl]
                    wc[par, 4 + j, sl] = tw[a, j, sl]
            for t in range(16):
                pltpu.async_copy(tbl.at[idx.at[par, t]], rows.at[par, t],
                                 sems[par])

        def wait_step(par, step):
            tbl = scales[step // 3][1][step % 3]
            for t in range(16):
                pltpu.make_async_copy(tbl.at[idx.at[par, t]],
                                      rows.at[par, t], sems[par]).wait()

        def accumulate(par, first):
            ones = jnp.ones((_L,), jnp.float32)

            def point(p, c2):
                wy = [wc[par, j, pl.ds(p, _L)][0] for j in range(4)]
                wx = [wc[par, 4 + i, pl.ds(p, _L)][0] for i in range(4)]
                wxp = [plsc.pack(w * ones, w * ones,
                                 format=plsc.PackFormat.INTERLEAVED)
                       for w in wx]
                for g in range(_C // (2 * _L)):
                    ga = pl.ds(g * 2 * _L, _L)
                    gb = pl.ds(g * 2 * _L + _L, _L)
                    ava = None if first else acc[p, ga]
                    avb = None if first else acc[p, gb]
                    for j in range(4):
                        bv = None
                        for i in range(4):
                            x = rows[par, j * 4 + i, p,
                                     pl.ds(g * 2 * _L, 2 * _L)]
                            t = wxp[i] * x
                            bv = t if bv is None else bv + t
                        a, b = plsc.unpack(
                            bv, format=plsc.PackFormat.INTERLEAVED)
                        pa, pb = wy[j] * a, wy[j] * b
                        ava = pa if ava is None else ava + pa
                        avb = pb if avb is None else avb + pb
                    acc[p, ga] = ava
                    acc[p, gb] = avb
                return c2
            lax.fori_loop(0, _B, point, 0)

        load_coords(0)
        comp_taps(scales[0][0])
        build_fire(0, 0)

        def chunk(ci, carry):
            base = start + ci * _B
            for k in range(9):
                par = k % _NBUF
                nxt = (k + 1) % _NBUF
                if k < 8:
                    if (k + 1) % 3 == 0:
                        comp_taps(scales[(k + 1) // 3][0])
                    build_fire(nxt, k + 1)
                else:
                    @pl.when(ci < nchunk - 1)
                    def _():
                        load_coords(ci + 1)
                        comp_taps(scales[0][0])
                        build_fire(nxt, 0)
                wait_step(par, k)
                accumulate(par, first=(k % 3 == 0))
                if k % 3 == 2:
                    col = scales[k // 3][2]
                    pltpu.sync_copy(
                        acc, out.at[pl.ds(base, _B), pl.ds(col, _C)])
            return carry
        lax.fori_loop(0, nchunk, chunk, 0)

    out_type = jax.ShapeDtypeStruct((P, 3 * _C), jnp.float32)
    scratch = [
        pltpu.VMEM((3, _B), jnp.float32),
        pltpu.VMEM((3, 4, _B), jnp.int32),
        pltpu.VMEM((3, 4, _B), jnp.float32),
        pltpu.VMEM((_NBUF, 16, _B), jnp.int32),
        pltpu.VMEM((_NBUF, 8, _B + _L), jnp.float32),
        pltpu.VMEM((_NBUF, 16, _B, _C), jnp.bfloat16),
        pltpu.VMEM((_B, _C), jnp.float32),
        pltpu.SemaphoreType.DMA,
        pltpu.SemaphoreType.DMA,
        pltpu.SemaphoreType.DMA,
    ]
    return pl.kernel(body, out_type=out_type, mesh=mesh,
                     scratch_types=scratch,
                     compiler_params=pltpu.CompilerParams(
                         use_tc_tiling_on_sc=False,
                         needs_layout_passes=False))


def kernel(coordinates, px1, py1, pz1, px2, py2, pz2, px3, py3, pz3):
    P = coordinates.shape[0]
    planes = (px1, py1, pz1, px2, py2, pz2, px3, py3, pz3)
    grids = (px1.shape[1], px2.shape[1], px3.shape[1])
    ct = coordinates.T
    perm = np.stack([np.arange(16), np.arange(16) + 16], 1).reshape(-1)
    perm = np.concatenate([perm, perm + 32])
    tables = [
        jnp.transpose(p, (1, 2, 0)).reshape(-1, _C)
        .astype(jnp.bfloat16)[:, perm]
        for p in planes]
    call = _make_sc_call(P, grids)
    return call(ct, *tables)


# speedup vs baseline: 1.1942x; 1.0035x over previous
import functools

import numpy as np

import jax
import jax.numpy as jnp
from jax import lax
from jax.experimental import pallas as pl
from jax.experimental.pallas import tpu as pltpu
from jax.experimental.pallas import tpu_sc as plsc

_NC = 2
_NS = 16
_L = 16
_NW = _NC * _NS
_C = 64
_B = 32
_NBUF = 3


def _cc1(x):
    return ((-0.75 + 2.0) * x - (-0.75 + 3.0)) * x * x + 1.0


def _cc2(x):
    return ((-0.75 * x - 5.0 * -0.75) * x + 8.0 * -0.75) * x - 4.0 * -0.75


def _make_sc_call(P, grids):
    pts_w = P // _NW
    nchunk = pts_w // _B
    mesh = plsc.VectorSubcoreMesh(
        core_axis_name="c", subcore_axis_name="s",
        num_cores=_NC, num_subcores=_NS)

    def body(ct, t1x, t1y, t1z, t2x, t2y, t2z, t3x, t3y, t3z,
             out, cb, ti, tw, idx, wc, rows, acc,
             sem0, sem1, sem2):
        wid = lax.axis_index("s") * _NC + lax.axis_index("c")
        start = wid * pts_w
        sems = (sem0, sem1, sem2)
        scales = ((grids[0], (t1x, t1y, t1z), 0),
                  (grids[1], (t2x, t2y, t2z), _C),
                  (grids[2], (t3x, t3y, t3z), 2 * _C))
        axes = ((0, 1), (1, 2), (0, 2))

        def load_coords(ci):
            base = start + ci * _B
            for k in range(3):
                pltpu.sync_copy(ct.at[k, pl.ds(base, _B)], cb.at[k])

        def comp_taps(W):
            for k in range(3):
                for v in range(_B // _L):
                    sl = pl.ds(v * _L, _L)
                    c = (cb[k, sl] + 1.0) * 0.5
                    ix = (c + 1.0) * 0.5 * (W - 1)
                    f = ix.astype(jnp.int32)
                    t = ix - f.astype(jnp.float32)
                    ws = (_cc2(t + 1.0), _cc1(t), _cc1(1.0 - t),
                          _cc2(2.0 - t))
                    for j in range(4):
                        ti[k, j, sl] = jnp.clip(f - 1 + j, 0, W - 1)
                        tw[k, j, sl] = ws[j]

        def build_fire(par, step):
            s, p = step // 3, step % 3
            W, tbls, _ = scales[s]
            a, b = axes[p]
            tbl = tbls[p]
            for j in range(4):
                for i in range(4):
                    t = j * 4 + i
                    for v in range(_B // _L):
                        sl = pl.ds(v * _L, _L)
                        idx[par, t, sl] = ti[b, j, sl] * W + ti[a, i, sl]
            for j in range(4):
                for v in range(_B // _L):
                    sl = pl.ds(v * _L, _L)
                    wc[par, j, sl] = tw[b, j, sl]
                    wc[par, 4 + j, sl] = tw[a, j, sl]
            for t in range(16):
                pltpu.async_copy(tbl.at[idx.at[par, t]], rows.at[par, t],
                                 sems[par])

        def wait_step(par, step):
            tbl = scales[step // 3][1][step % 3]
            for t in range(16):
                pltpu.make_async_copy(tbl.at[idx.at[par, t]],
                                      rows.at[par, t], sems[par]).wait()

        def accumulate(par, first):
            ones = jnp.ones((_L,), jnp.float32)

            def point(p, c2):
                wy = [wc[par, j, pl.ds(p, _L)][0] for j in range(4)]
                wx = [wc[par, 4 + i, pl.ds(p, _L)][0] for i in range(4)]
                wxp = [plsc.pack(w * ones, w * ones,
                                 format=plsc.PackFormat.INTERLEAVED)
                       for w in wx]
                for g in range(_C // (2 * _L)):
                    ga = pl.ds(g * 2 * _L, _L)
                    gb = pl.ds(g * 2 * _L + _L, _L)
                    ava = avb = None
                    for j in range(4):
                        bv = None
                        for i in range(4):
                            x = rows[par, j * 4 + i, p,
                                     pl.ds(g * 2 * _L, 2 * _L)]
                            t = wxp[i] * x
                            bv = t if bv is None else bv + t
                        a, b = plsc.unpack(
                            bv, format=plsc.PackFormat.INTERLEAVED)
                        pa, pb = wy[j] * a, wy[j] * b
                        ava = pa if ava is None else ava + pa
                        avb = pb if avb is None else avb + pb
                    if first:
                        acc[p, ga] = ava
                        acc[p, gb] = avb
                    else:
                        plsc.addupdate(acc.at[p, ga], ava)
                        plsc.addupdate(acc.at[p, gb], avb)
                return c2
            lax.fori_loop(0, _B, point, 0)

        load_coords(0)
        comp_taps(scales[0][0])
        build_fire(0, 0)

        def chunk(ci, carry):
            base = start + ci * _B
            for k in range(9):
                par = k % _NBUF
                nxt = (k + 1) % _NBUF
                if k < 8:
                    if (k + 1) % 3 == 0:
                        comp_taps(scales[(k + 1) // 3][0])
                    build_fire(nxt, k + 1)
                else:
                    @pl.when(ci < nchunk - 1)
                    def _():
                        load_coords(ci + 1)
                        comp_taps(scales[0][0])
                        build_fire(nxt, 0)
                wait_step(par, k)
                accumulate(par, first=(k % 3 == 0))
                if k % 3 == 2:
                    col = scales[k // 3][2]
                    pltpu.sync_copy(
                        acc, out.at[pl.ds(base, _B), pl.ds(col, _C)])
            return carry
        lax.fori_loop(0, nchunk, chunk, 0)

    out_type = jax.ShapeDtypeStruct((P, 3 * _C), jnp.float32)
    scratch = [
        pltpu.VMEM((3, _B), jnp.float32),
        pltpu.VMEM((3, 4, _B), jnp.int32),
        pltpu.VMEM((3, 4, _B), jnp.float32),
        pltpu.VMEM((_NBUF, 16, _B), jnp.int32),
        pltpu.VMEM((_NBUF, 8, _B + _L), jnp.float32),
        pltpu.VMEM((_NBUF, 16, _B, _C), jnp.bfloat16),
        pltpu.VMEM((_B, _C), jnp.float32),
        pltpu.SemaphoreType.DMA,
        pltpu.SemaphoreType.DMA,
        pltpu.SemaphoreType.DMA,
    ]
    return pl.kernel(body, out_type=out_type, mesh=mesh,
                     scratch_types=scratch,
                     compiler_params=pltpu.CompilerParams(
                         use_tc_tiling_on_sc=False,
                         needs_layout_passes=False))


def kernel(coordinates, px1, py1, pz1, px2, py2, pz2, px3, py3, pz3):
    P = coordinates.shape[0]
    planes = (px1, py1, pz1, px2, py2, pz2, px3, py3, pz3)
    grids = (px1.shape[1], px2.shape[1], px3.shape[1])
    ct = coordinates.T
    perm = np.stack([np.arange(16), np.arange(16) + 16], 1).reshape(-1)
    perm = np.concatenate([perm, perm + 32])
    tables = [
        jnp.transpose(p, (1, 2, 0)).reshape(-1, _C)
        .astype(jnp.bfloat16)[:, perm]
        for p in planes]
    call = _make_sc_call(P, grids)
    return call(ct, *tables)


# speedup vs baseline: 1.2537x; 1.0499x over previous
import functools

import numpy as np

import jax
import jax.numpy as jnp
from jax import lax
from jax.experimental import pallas as pl
from jax.experimental.pallas import tpu as pltpu
from jax.experimental.pallas import tpu_sc as plsc

_NC = 2
_NS = 16
_L = 16
_NW = _NC * _NS
_C = 64
_B = 64
_NBUF = 3


def _cc1(x):
    return ((-0.75 + 2.0) * x - (-0.75 + 3.0)) * x * x + 1.0


def _cc2(x):
    return ((-0.75 * x - 5.0 * -0.75) * x + 8.0 * -0.75) * x - 4.0 * -0.75


def _make_sc_call(P, grids):
    pts_w = P // _NW
    nchunk = pts_w // _B
    mesh = plsc.VectorSubcoreMesh(
        core_axis_name="c", subcore_axis_name="s",
        num_cores=_NC, num_subcores=_NS)

    def body(ct, t1x, t1y, t1z, t2x, t2y, t2z, t3x, t3y, t3z,
             out, cb, ti, tw, idx, wc, rows, acc,
             sem0, sem1, sem2):
        wid = lax.axis_index("s") * _NC + lax.axis_index("c")
        start = wid * pts_w
        sems = (sem0, sem1, sem2)
        scales = ((grids[0], (t1x, t1y, t1z), 0),
                  (grids[1], (t2x, t2y, t2z), _C),
                  (grids[2], (t3x, t3y, t3z), 2 * _C))
        axes = ((0, 1), (1, 2), (0, 2))

        def load_coords(ci):
            base = start + ci * _B
            for k in range(3):
                pltpu.sync_copy(ct.at[k, pl.ds(base, _B)], cb.at[k])

        def comp_taps(W):
            for k in range(3):
                for v in range(_B // _L):
                    sl = pl.ds(v * _L, _L)
                    c = (cb[k, sl] + 1.0) * 0.5
                    ix = (c + 1.0) * 0.5 * (W - 1)
                    f = ix.astype(jnp.int32)
                    t = ix - f.astype(jnp.float32)
                    ws = (_cc2(t + 1.0), _cc1(t), _cc1(1.0 - t),
                          _cc2(2.0 - t))
                    for j in range(4):
                        ti[k, j, sl] = jnp.clip(f - 1 + j, 0, W - 1)
                        tw[k, j, sl] = ws[j]

        def build_fire(par, step):
            s, p = step // 3, step % 3
            W, tbls, _ = scales[s]
            a, b = axes[p]
            tbl = tbls[p]
            for j in range(4):
                for i in range(4):
                    t = j * 4 + i
                    for v in range(_B // _L):
                        sl = pl.ds(v * _L, _L)
                        idx[par, t, sl] = ti[b, j, sl] * W + ti[a, i, sl]
            for j in range(4):
                for v in range(_B // _L):
                    sl = pl.ds(v * _L, _L)
                    wc[par, j, sl] = tw[b, j, sl]
                    wc[par, 4 + j, sl] = tw[a, j, sl]
            for t in range(16):
                pltpu.async_copy(tbl.at[idx.at[par, t]], rows.at[par, t],
                                 sems[par])

        def wait_step(par, step):
            tbl = scales[step // 3][1][step % 3]
            for t in range(16):
                pltpu.make_async_copy(tbl.at[idx.at[par, t]],
                                      rows.at[par, t], sems[par]).wait()

        def accumulate(par, first):
            ones = jnp.ones((_L,), jnp.float32)

            def point(p, c2):
                wy = [wc[par, j, pl.ds(p, _L)][0] for j in range(4)]
                wx = [wc[par, 4 + i, pl.ds(p, _L)][0] for i in range(4)]
                wxp = [plsc.pack(w * ones, w * ones,
                                 format=plsc.PackFormat.INTERLEAVED)
                       for w in wx]
                for g in range(_C // (2 * _L)):
                    ga = pl.ds(g * 2 * _L, _L)
                    gb = pl.ds(g * 2 * _L + _L, _L)
                    ava = avb = None
                    for j in range(4):
                        bv = None
                        for i in range(4):
                            x = rows[par, j * 4 + i, p,
                                     pl.ds(g * 2 * _L, 2 * _L)]
                            t = wxp[i] * x
                            bv = t if bv is None else bv + t
                        a, b = plsc.unpack(
                            bv, format=plsc.PackFormat.INTERLEAVED)
                        pa, pb = wy[j] * a, wy[j] * b
                        ava = pa if ava is None else ava + pa
                        avb = pb if avb is None else avb + pb
                    if first:
                        acc[p, ga] = ava
                        acc[p, gb] = avb
                    else:
                        plsc.addupdate(acc.at[p, ga], ava)
                        plsc.addupdate(acc.at[p, gb], avb)
                return c2
            lax.fori_loop(0, _B, point, 0)

        load_coords(0)
        comp_taps(scales[0][0])
        build_fire(0, 0)

        def chunk(ci, carry):
            base = start + ci * _B
            for k in range(9):
                par = k % _NBUF
                nxt = (k + 1) % _NBUF
                if k < 8:
                    if (k + 1) % 3 == 0:
                        comp_taps(scales[(k + 1) // 3][0])
                    build_fire(nxt, k + 1)
                else:
                    @pl.when(ci < nchunk - 1)
                    def _():
                        load_coords(ci + 1)
                        comp_taps(scales[0][0])
                        build_fire(nxt, 0)
                wait_step(par, k)
                accumulate(par, first=(k % 3 == 0))
                if k % 3 == 2:
                    col = scales[k // 3][2]
                    pltpu.sync_copy(
                        acc, out.at[pl.ds(base, _B), pl.ds(col, _C)])
            return carry
        lax.fori_loop(0, nchunk, chunk, 0)

    out_type = jax.ShapeDtypeStruct((P, 3 * _C), jnp.float32)
    scratch = [
        pltpu.VMEM((3, _B), jnp.float32),
        pltpu.VMEM((3, 4, _B), jnp.int32),
        pltpu.VMEM((3, 4, _B), jnp.float32),
        pltpu.VMEM((_NBUF, 16, _B), jnp.int32),
        pltpu.VMEM((_NBUF, 8, _B + _L), jnp.float32),
        pltpu.VMEM((_NBUF, 16, _B, _C), jnp.bfloat16),
        pltpu.VMEM((_B, _C), jnp.float32),
        pltpu.SemaphoreType.DMA,
        pltpu.SemaphoreType.DMA,
        pltpu.SemaphoreType.DMA,
    ]
    return pl.kernel(body, out_type=out_type, mesh=mesh,
                     scratch_types=scratch,
                     compiler_params=pltpu.CompilerParams(
                         use_tc_tiling_on_sc=False,
                         needs_layout_passes=False))


def kernel(coordinates, px1, py1, pz1, px2, py2, pz2, px3, py3, pz3):
    P = coordinates.shape[0]
    planes = (px1, py1, pz1, px2, py2, pz2, px3, py3, pz3)
    grids = (px1.shape[1], px2.shape[1], px3.shape[1])
    ct = coordinates.T
    perm = np.stack([np.arange(16), np.arange(16) + 16], 1).reshape(-1)
    perm = np.concatenate([perm, perm + 32])
    tables = [
        jnp.transpose(p, (1, 2, 0)).reshape(-1, _C)
        .astype(jnp.bfloat16)[:, perm]
        for p in planes]
    call = _make_sc_call(P, grids)
    return call(ct, *tables)
